# baseline (device time: 31113 ns/iter reference)
import jax
import jax.numpy as jnp
from jax import lax
from jax.experimental import pallas as pl
from jax.experimental.pallas import tpu as pltpu

N_CHUNKS = 8
VIA_X = (0, 2, 4)
VIA_Z = (1, 3, 5)
VIA_Y = (6, 7)


def kernel(x):
    _, m, n_total = x.shape
    n_out = n_total // 2
    quarter = m // 4
    chunk = quarter // N_CHUNKS

    def body(x_ref, out_ref, xloc, ybuf, ydbuf, xdbuf, zdbuf, xrbuf, zrbuf,
             locsem, ysend, yrecv, ydsend, ydrecv, xdsend, xdrecv,
             zdsend, zdrecv, xrsend, xrrecv, zrsend, zrrecv):
        my_x = lax.axis_index("x")
        my_y = lax.axis_index("y")
        my_z = lax.axis_index("z")
        peer = (my_x, 1 - my_y, my_z)
        xnb = (1 - my_x, my_y, my_z)
        znb = (my_x, my_y, 1 - my_z)

        my_cols = pl.ds(my_y * n_out, n_out)
        peer_cols = pl.ds((1 - my_y) * n_out, n_out)

        local = pltpu.make_async_copy(
            x_ref.at[0, :, my_cols], xloc, locsem,
        )
        local.start()

        barrier_sem = pltpu.get_barrier_semaphore()
        for nbr in (peer, xnb, znb):
            pl.semaphore_signal(
                barrier_sem, inc=1,
                device_id=nbr, device_id_type=pl.DeviceIdType.MESH,
            )
        pl.semaphore_wait(barrier_sem, 3)
        q_mine = (2 * my_x + my_z) * quarter
        q_xnb = (2 * (1 - my_x) + my_z) * quarter
        q_znb = (2 * my_x + (1 - my_z)) * quarter
        q_diag = (2 * (1 - my_x) + (1 - my_z)) * quarter

        def cs(base, c):
            return pl.ds(base + c * chunk, chunk)

        y_rdmas = []
        for c in range(N_CHUNKS):
            r = pltpu.make_async_remote_copy(
                src_ref=x_ref.at[0, cs(q_mine, c), peer_cols],
                dst_ref=ybuf.at[cs(0, c), :],
                send_sem=ysend.at[c],
                recv_sem=yrecv.at[c],
                device_id=peer,
                device_id_type=pl.DeviceIdType.MESH,
            )
            r.start()
            y_rdmas.append(r)
        yd_rdmas = []
        for j, c in enumerate(VIA_Y):
            r = pltpu.make_async_remote_copy(
                src_ref=x_ref.at[0, cs(q_diag, c), peer_cols],
                dst_ref=ydbuf.at[cs(0, j), :],
                send_sem=ydsend.at[j],
                recv_sem=ydrecv.at[j],
                device_id=peer,
                device_id_type=pl.DeviceIdType.MESH,
            )
            r.start()
            yd_rdmas.append(r)

        local.wait()
        xd_rdmas, zd_rdmas = [], []
        for c in range(N_CHUNKS):
            y_rdmas[c].wait_recv()
            rx = pltpu.make_async_remote_copy(
                src_ref=ybuf.at[cs(0, c), :],
                dst_ref=xdbuf.at[cs(0, c), :],
                send_sem=xdsend.at[c],
                recv_sem=xdrecv.at[c],
                device_id=xnb,
                device_id_type=pl.DeviceIdType.MESH,
            )
            rx.start()
            xd_rdmas.append(rx)
            rz = pltpu.make_async_remote_copy(
                src_ref=ybuf.at[cs(0, c), :],
                dst_ref=zdbuf.at[cs(0, c), :],
                send_sem=zdsend.at[c],
                recv_sem=zdrecv.at[c],
                device_id=znb,
                device_id_type=pl.DeviceIdType.MESH,
            )
            rz.start()
            zd_rdmas.append(rz)
            out_ref[cs(q_mine, c), :] = (
                xloc[cs(q_mine, c), :] + ybuf[cs(0, c), :]
            )

        xr_rdmas, zr_rdmas = [], []
        for c in range(N_CHUNKS):
            zd_rdmas[c].wait_recv()
            if c in VIA_X:
                j = VIA_X.index(c)
                rr = pltpu.make_async_remote_copy(
                    src_ref=zdbuf.at[cs(0, c), :],
                    dst_ref=xrbuf.at[cs(0, j), :],
                    send_sem=xrsend.at[j],
                    recv_sem=xrrecv.at[j],
                    device_id=xnb,
                    device_id_type=pl.DeviceIdType.MESH,
                )
                rr.start()
                xr_rdmas.append(rr)
            out_ref[cs(q_znb, c), :] = (
                xloc[cs(q_znb, c), :] + zdbuf[cs(0, c), :]
            )
            xd_rdmas[c].wait_recv()
            if c in VIA_Z:
                j = VIA_Z.index(c)
                rr = pltpu.make_async_remote_copy(
                    src_ref=xdbuf.at[cs(0, c), :],
                    dst_ref=zrbuf.at[cs(0, j), :],
                    send_sem=zrsend.at[j],
                    recv_sem=zrrecv.at[j],
                    device_id=znb,
                    device_id_type=pl.DeviceIdType.MESH,
                )
                rr.start()
                zr_rdmas.append(rr)
            out_ref[cs(q_xnb, c), :] = (
                xloc[cs(q_xnb, c), :] + xdbuf[cs(0, c), :]
            )

        for j in range(len(VIA_X)):
            xr_rdmas[j].wait_recv()
            c = VIA_X[j]
            out_ref[cs(q_diag, c), :] = (
                xloc[cs(q_diag, c), :] + xrbuf[cs(0, j), :]
            )
            zr_rdmas[j].wait_recv()
            c = VIA_Z[j]
            out_ref[cs(q_diag, c), :] = (
                xloc[cs(q_diag, c), :] + zrbuf[cs(0, j), :]
            )
        for j in range(len(VIA_Y)):
            yd_rdmas[j].wait_recv()
            c = VIA_Y[j]
            out_ref[cs(q_diag, c), :] = (
                xloc[cs(q_diag, c), :] + ydbuf[cs(0, j), :]
            )

        for c in range(N_CHUNKS):
            y_rdmas[c].wait_send()
            xd_rdmas[c].wait_send()
            zd_rdmas[c].wait_send()
        for j in range(len(VIA_X)):
            xr_rdmas[j].wait_send()
            zr_rdmas[j].wait_send()
        for j in range(len(VIA_Y)):
            yd_rdmas[j].wait_send()

    n_x = len(VIA_X)
    n_z = len(VIA_Z)
    n_y = len(VIA_Y)
    return pl.pallas_call(
        body,
        out_shape=jax.ShapeDtypeStruct((m, n_out), x.dtype),
        in_specs=[pl.BlockSpec(memory_space=pl.ANY)],
        out_specs=pl.BlockSpec(memory_space=pltpu.VMEM),
        scratch_shapes=[
            pltpu.VMEM((m, n_out), x.dtype),
            pltpu.VMEM((quarter, n_out), x.dtype),
            pltpu.VMEM((n_y * chunk, n_out), x.dtype),
            pltpu.VMEM((quarter, n_out), x.dtype),
            pltpu.VMEM((quarter, n_out), x.dtype),
            pltpu.VMEM((n_x * chunk, n_out), x.dtype),
            pltpu.VMEM((n_z * chunk, n_out), x.dtype),
            pltpu.SemaphoreType.DMA,
            pltpu.SemaphoreType.DMA((N_CHUNKS,)),
            pltpu.SemaphoreType.DMA((N_CHUNKS,)),
            pltpu.SemaphoreType.DMA((n_y,)),
            pltpu.SemaphoreType.DMA((n_y,)),
            pltpu.SemaphoreType.DMA((N_CHUNKS,)),
            pltpu.SemaphoreType.DMA((N_CHUNKS,)),
            pltpu.SemaphoreType.DMA((N_CHUNKS,)),
            pltpu.SemaphoreType.DMA((N_CHUNKS,)),
            pltpu.SemaphoreType.DMA((n_x,)),
            pltpu.SemaphoreType.DMA((n_x,)),
            pltpu.SemaphoreType.DMA((n_z,)),
            pltpu.SemaphoreType.DMA((n_z,)),
        ],
        compiler_params=pltpu.CompilerParams(collective_id=0),
    )(x)


# device time: 28844 ns/iter; 1.0787x vs baseline; 1.0787x over previous
import jax
import jax.numpy as jnp
from jax import lax
from jax.experimental import pallas as pl
from jax.experimental.pallas import tpu as pltpu

N_CHUNKS = 16
VIA_X = (0, 2, 4, 6)
VIA_Z = (1, 3, 5, 7)
VIA_Y = (8, 9, 10, 11, 12, 13, 14, 15)


def kernel(x):
    _, m, n_total = x.shape
    n_out = n_total // 2
    quarter = m // 4
    chunk = quarter // N_CHUNKS

    def body(x_ref, out_ref, ybuf, ydbuf, xdbuf, zdbuf, xrbuf, zrbuf,
             ysend, yrecv, ydsend, ydrecv, xdsend, xdrecv,
             zdsend, zdrecv, xrsend, xrrecv, zrsend, zrrecv):
        my_x = lax.axis_index("x")
        my_y = lax.axis_index("y")
        my_z = lax.axis_index("z")
        peer = (my_x, 1 - my_y, my_z)
        xnb = (1 - my_x, my_y, my_z)
        znb = (my_x, my_y, 1 - my_z)

        my_cols = pl.ds(my_y * n_out, n_out)
        peer_cols = pl.ds((1 - my_y) * n_out, n_out)

        barrier_sem = pltpu.get_barrier_semaphore()
        for nbr in (peer, xnb, znb):
            pl.semaphore_signal(
                barrier_sem, inc=1,
                device_id=nbr, device_id_type=pl.DeviceIdType.MESH,
            )
        pl.semaphore_wait(barrier_sem, 3)
        q_mine = (2 * my_x + my_z) * quarter
        q_xnb = (2 * (1 - my_x) + my_z) * quarter
        q_znb = (2 * my_x + (1 - my_z)) * quarter
        q_diag = (2 * (1 - my_x) + (1 - my_z)) * quarter

        def cs(base, c):
            return pl.ds(base + c * chunk, chunk)

        y_rdmas = []
        for c in range(N_CHUNKS):
            r = pltpu.make_async_remote_copy(
                src_ref=x_ref.at[0, cs(q_mine, c), peer_cols],
                dst_ref=ybuf.at[cs(0, c), :],
                send_sem=ysend.at[c],
                recv_sem=yrecv.at[c],
                device_id=peer,
                device_id_type=pl.DeviceIdType.MESH,
            )
            r.start()
            y_rdmas.append(r)
        yd_rdmas = []
        for j, c in enumerate(VIA_Y):
            r = pltpu.make_async_remote_copy(
                src_ref=x_ref.at[0, cs(q_diag, c), peer_cols],
                dst_ref=ydbuf.at[cs(0, j), :],
                send_sem=ydsend.at[j],
                recv_sem=ydrecv.at[j],
                device_id=peer,
                device_id_type=pl.DeviceIdType.MESH,
            )
            r.start()
            yd_rdmas.append(r)

        xd_rdmas, zd_rdmas = [], []
        for c in range(N_CHUNKS):
            y_rdmas[c].wait_recv()
            rx = pltpu.make_async_remote_copy(
                src_ref=ybuf.at[cs(0, c), :],
                dst_ref=xdbuf.at[cs(0, c), :],
                send_sem=xdsend.at[c],
                recv_sem=xdrecv.at[c],
                device_id=xnb,
                device_id_type=pl.DeviceIdType.MESH,
            )
            rx.start()
            xd_rdmas.append(rx)
            rz = pltpu.make_async_remote_copy(
                src_ref=ybuf.at[cs(0, c), :],
                dst_ref=zdbuf.at[cs(0, c), :],
                send_sem=zdsend.at[c],
                recv_sem=zdrecv.at[c],
                device_id=znb,
                device_id_type=pl.DeviceIdType.MESH,
            )
            rz.start()
            zd_rdmas.append(rz)
            out_ref[cs(q_mine, c), :] = (
                x_ref[0, cs(q_mine, c), my_cols] + ybuf[cs(0, c), :]
            )

        xr_rdmas, zr_rdmas = [], []
        for c in range(N_CHUNKS):
            zd_rdmas[c].wait_recv()
            if c in VIA_X:
                j = VIA_X.index(c)
                rr = pltpu.make_async_remote_copy(
                    src_ref=zdbuf.at[cs(0, c), :],
                    dst_ref=xrbuf.at[cs(0, j), :],
                    send_sem=xrsend.at[j],
                    recv_sem=xrrecv.at[j],
                    device_id=xnb,
                    device_id_type=pl.DeviceIdType.MESH,
                )
                rr.start()
                xr_rdmas.append(rr)
            out_ref[cs(q_znb, c), :] = (
                x_ref[0, cs(q_znb, c), my_cols] + zdbuf[cs(0, c), :]
            )
            xd_rdmas[c].wait_recv()
            if c in VIA_Z:
                j = VIA_Z.index(c)
                rr = pltpu.make_async_remote_copy(
                    src_ref=xdbuf.at[cs(0, c), :],
                    dst_ref=zrbuf.at[cs(0, j), :],
                    send_sem=zrsend.at[j],
                    recv_sem=zrrecv.at[j],
                    device_id=znb,
                    device_id_type=pl.DeviceIdType.MESH,
                )
                rr.start()
                zr_rdmas.append(rr)
            out_ref[cs(q_xnb, c), :] = (
                x_ref[0, cs(q_xnb, c), my_cols] + xdbuf[cs(0, c), :]
            )

        for j in range(len(VIA_X)):
            xr_rdmas[j].wait_recv()
            c = VIA_X[j]
            out_ref[cs(q_diag, c), :] = (
                x_ref[0, cs(q_diag, c), my_cols] + xrbuf[cs(0, j), :]
            )
        for j in range(len(VIA_Z)):
            zr_rdmas[j].wait_recv()
            c = VIA_Z[j]
            out_ref[cs(q_diag, c), :] = (
                x_ref[0, cs(q_diag, c), my_cols] + zrbuf[cs(0, j), :]
            )
        for j in range(len(VIA_Y)):
            yd_rdmas[j].wait_recv()
            c = VIA_Y[j]
            out_ref[cs(q_diag, c), :] = (
                x_ref[0, cs(q_diag, c), my_cols] + ydbuf[cs(0, j), :]
            )

        for c in range(N_CHUNKS):
            y_rdmas[c].wait_send()
            xd_rdmas[c].wait_send()
            zd_rdmas[c].wait_send()
        for j in range(len(VIA_X)):
            xr_rdmas[j].wait_send()
        for j in range(len(VIA_Z)):
            zr_rdmas[j].wait_send()
        for j in range(len(VIA_Y)):
            yd_rdmas[j].wait_send()

    n_x = len(VIA_X)
    n_z = len(VIA_Z)
    n_y = len(VIA_Y)
    return pl.pallas_call(
        body,
        out_shape=jax.ShapeDtypeStruct((m, n_out), x.dtype),
        in_specs=[pl.BlockSpec(memory_space=pltpu.VMEM)],
        out_specs=pl.BlockSpec(memory_space=pltpu.VMEM),
        scratch_shapes=[
            pltpu.VMEM((quarter, n_out), x.dtype),
            pltpu.VMEM((n_y * chunk, n_out), x.dtype),
            pltpu.VMEM((quarter, n_out), x.dtype),
            pltpu.VMEM((quarter, n_out), x.dtype),
            pltpu.VMEM((n_x * chunk, n_out), x.dtype),
            pltpu.VMEM((n_z * chunk, n_out), x.dtype),
            pltpu.SemaphoreType.DMA((N_CHUNKS,)),
            pltpu.SemaphoreType.DMA((N_CHUNKS,)),
            pltpu.SemaphoreType.DMA((n_y,)),
            pltpu.SemaphoreType.DMA((n_y,)),
            pltpu.SemaphoreType.DMA((N_CHUNKS,)),
            pltpu.SemaphoreType.DMA((N_CHUNKS,)),
            pltpu.SemaphoreType.DMA((N_CHUNKS,)),
            pltpu.SemaphoreType.DMA((N_CHUNKS,)),
            pltpu.SemaphoreType.DMA((n_x,)),
            pltpu.SemaphoreType.DMA((n_x,)),
            pltpu.SemaphoreType.DMA((n_z,)),
            pltpu.SemaphoreType.DMA((n_z,)),
        ],
        compiler_params=pltpu.CompilerParams(collective_id=0),
    )(x)
